# baseline (speedup 1.0000x reference)
import jax
import jax.numpy as jnp
from jax.experimental import pallas as pl

_K = 20
_B = 128
_N = 8192


def _rank_kernel(z_ref, labels_ref, ones_ref):
    za = jnp.abs(z_ref[...])
    iota = jax.lax.broadcasted_iota(jnp.int32, (_B, _N), 1)

    def body(_, carry):
        za, mask = carry
        m = jnp.max(za, axis=1, keepdims=True)
        is_m = za == m
        im = jnp.min(jnp.where(is_m, iota, _N), axis=1, keepdims=True)
        sel = iota == im
        mask = jnp.where(sel, jnp.float32(1.0), mask)
        za = jnp.where(sel, jnp.float32(-1.0), za)
        return za, mask

    _, mask = jax.lax.fori_loop(
        0, _K, body, (za, jnp.zeros((_B, _N), jnp.float32))
    )
    mb = mask.astype(jnp.bfloat16)
    g = jax.lax.dot_general(
        mb, mb, (((1,), (1,)), ((), ())), preferred_element_type=jnp.float32
    )
    labels_ref[...] = (g > _K - 0.5).astype(jnp.float32)
    ones_ref[...] = jnp.ones((_B, _B), jnp.float32)


def kernel(z):
    labels, ones = pl.pallas_call(
        _rank_kernel,
        out_shape=(
            jax.ShapeDtypeStruct((_B, _B), jnp.float32),
            jax.ShapeDtypeStruct((_B, _B), jnp.float32),
        ),
    )(z)
    return labels, ones


# speedup vs baseline: 2.0262x; 2.0262x over previous
import jax
import jax.numpy as jnp
from jax.experimental import pallas as pl

_K = 20
_B = 128
_N = 8192


def _rank_kernel(z_ref, labels_ref, ones_ref):
    bits = jax.lax.bitcast_convert_type(z_ref[...], jnp.int32) & jnp.int32(
        0x7FFFFFFF
    )
    iota = jax.lax.broadcasted_iota(jnp.int32, (_B, _N), 1)

    def count_ge(c):
        ge = (bits >= c).astype(jnp.int32)
        return jnp.sum(ge, axis=1, keepdims=True)

    def vbody(i, t):
        cand = t | jax.lax.shift_left(jnp.int32(1), jnp.int32(30) - i)
        ok = count_ge(cand) >= _K
        return jnp.where(ok, cand, t)

    t = jax.lax.fori_loop(0, 31, vbody, jnp.zeros((_B, 1), jnp.int32))

    gt = bits > t
    eq = bits == t
    cnt_gt = jnp.sum(gt.astype(jnp.int32), axis=1, keepdims=True)

    def ibody(i, cur):
        cand = cur + jax.lax.shift_left(jnp.int32(1), jnp.int32(13) - i)
        cnt = cnt_gt + jnp.sum(
            (eq & (iota < cand)).astype(jnp.int32), axis=1, keepdims=True
        )
        ok = (cnt <= _K - 1) & (cand <= _N)
        return jnp.where(ok, cand, cur)

    cut = jax.lax.fori_loop(0, 14, ibody, jnp.zeros((_B, 1), jnp.int32))

    sel = gt.astype(jnp.int32) + eq.astype(jnp.int32) * (
        iota <= cut
    ).astype(jnp.int32)
    mb = sel.astype(jnp.bfloat16)
    g = jax.lax.dot_general(
        mb, mb, (((1,), (1,)), ((), ())), preferred_element_type=jnp.float32
    )
    labels_ref[...] = (g > _K - 0.5).astype(jnp.float32)
    ones_ref[...] = jnp.ones((_B, _B), jnp.float32)


def kernel(z):
    labels, ones = pl.pallas_call(
        _rank_kernel,
        out_shape=(
            jax.ShapeDtypeStruct((_B, _B), jnp.float32),
            jax.ShapeDtypeStruct((_B, _B), jnp.float32),
        ),
    )(z)
    return labels, ones
